# baseline (speedup 1.0000x reference)
import math

import jax
import jax.numpy as jnp
from jax.experimental import pallas as pl
from jax.experimental.pallas import tpu as pltpu

N_ANCHORS = 20000
PRE_K = 2000
KPAD = 2048
BLK = 256
NBLK = KPAD // BLK
POST_K = 1000
OUTPAD = 1024
NMS_T = 0.7
IMG = 1024.0
CLAMP = math.log(1000.0 / 16.0)


def _decode_kernel(a_ref, d_ref, s_ref, boxes_ref, ms_ref):
    at = jnp.transpose(a_ref[...])
    dt = jnp.transpose(d_ref[...])
    x1a, y1a, x2a, y2a = (at[0:1, :], at[1:2, :], at[2:3, :], at[3:4, :])
    dx, dy, dw, dh = (dt[0:1, :], dt[1:2, :], dt[2:3, :], dt[3:4, :])
    w = x2a - x1a
    h = y2a - y1a
    cx = x1a + 0.5 * w
    cy = y1a + 0.5 * h
    dw = jnp.minimum(dw, CLAMP)
    dh = jnp.minimum(dh, CLAMP)
    px = dx * w + cx
    py = dy * h + cy
    pw = jnp.exp(dw) * w
    ph = jnp.exp(dh) * h
    x1 = jnp.clip(px - 0.5 * pw, 0.0, IMG)
    y1 = jnp.clip(py - 0.5 * ph, 0.0, IMG)
    x2 = jnp.clip(px + 0.5 * pw, 0.0, IMG)
    y2 = jnp.clip(py + 0.5 * ph, 0.0, IMG)
    out_t = jnp.concatenate([x1, y1, x2, y2], axis=0)
    boxes_ref[...] = jnp.transpose(out_t)
    valid = ((x2 - x1) > 0.0) & ((y2 - y1) > 0.0)
    ms_ref[...] = jnp.where(valid, s_ref[...], -jnp.inf)


def _iou_block(cols, rows):
    x1i, y1i, x2i, y2i = cols
    x1j, y1j, x2j, y2j = rows
    ai = (x2i - x1i) * (y2i - y1i)
    aj = (x2j - x1j) * (y2j - y1j)
    wx = jnp.clip(jnp.minimum(x2i, x2j) - jnp.maximum(x1i, x1j), 0.0, None)
    wy = jnp.clip(jnp.minimum(y2i, y2j) - jnp.maximum(y1i, y1j), 0.0, None)
    inter = wx * wy
    union = ai + aj - inter
    return inter / jnp.maximum(union, 1e-9)


def _nms_kernel(in_tb_ref, in_sc_ref, out_ref, tb_ref, tbt_ref, over_ref):
    f32 = jnp.float32
    tb_ref[0:PRE_K, :] = in_tb_ref[...]
    tb_ref[PRE_K:KPAD, :] = jnp.zeros((KPAD - PRE_K, 4), f32)
    tbt_ref[...] = jnp.transpose(tb_ref[...])

    upper_s = (jax.lax.broadcasted_iota(jnp.int32, (BLK, BLK), 0)
               < jax.lax.broadcasted_iota(jnp.int32, (BLK, BLK), 1)).astype(f32)

    def cols_of(i):
        return tuple(tb_ref[i * BLK:(i + 1) * BLK, c:c + 1] for c in range(4))

    def rows_of(j):
        return tuple(tbt_ref[c:c + 1, j * BLK:(j + 1) * BLK] for c in range(4))

    keep_blocks = []
    for j in range(NBLK):
        rows_j = rows_of(j)
        supp = jnp.zeros((1, BLK), f32)
        for i in range(j):
            over_ij = (_iou_block(cols_of(i), rows_j) > NMS_T).astype(f32)
            supp = supp + jnp.dot(keep_blocks[i], over_ij,
                                  preferred_element_type=f32)
        over_ref[...] = ((_iou_block(cols_of(j), rows_j) > NMS_T).astype(f32)
                         * upper_s)

        def fix_body(carry):
            kb, _ = carry
            s2 = supp + jnp.dot(kb, over_ref[...], preferred_element_type=f32)
            kb2 = (s2 < 0.5).astype(f32)
            return kb2, jnp.any(kb2 != kb)

        keep_j, _ = jax.lax.while_loop(
            lambda c: c[1], fix_body,
            ((supp < 0.5).astype(f32), jnp.bool_(True)))
        keep_blocks.append(keep_j)

    keep = jnp.concatenate(keep_blocks, axis=0)
    gi = (jax.lax.broadcasted_iota(jnp.int32, (NBLK, BLK), 0) * BLK
          + jax.lax.broadcasted_iota(jnp.int32, (NBLK, BLK), 1))
    real = (gi < PRE_K).astype(f32)
    keep = keep * real
    nonkeep = real - keep

    upper = (jax.lax.broadcasted_iota(jnp.int32, (BLK, BLK), 0)
             <= jax.lax.broadcasted_iota(jnp.int32, (BLK, BLK), 1)).astype(f32)
    incl_k = jnp.dot(keep, upper, preferred_element_type=f32)
    incl_n = jnp.dot(nonkeep, upper, preferred_element_type=f32)

    def row_offsets(incl):
        offs = [jnp.zeros((1, 1), f32)]
        for r in range(1, NBLK):
            offs.append(offs[-1] + incl[r - 1:r, BLK - 1:BLK])
        return jnp.concatenate(offs, axis=0)

    offs_k = row_offsets(incl_k)
    offs_n = row_offsets(incl_n)
    n_kept = offs_k[NBLK - 1:NBLK, 0:1] + incl_k[NBLK - 1:NBLK, BLK - 1:BLK]
    excl_k = incl_k + offs_k - keep
    excl_n = incl_n + offs_n - nonkeep
    pos = jnp.where(keep > 0.5, excl_k, n_kept + excl_n)
    pos = jnp.where(gi < PRE_K, pos, 2.0 * OUTPAD)

    sc = in_sc_ref[...]
    neg_inf_flag = jnp.concatenate(
        [(sc == -jnp.inf).astype(f32), jnp.ones((KPAD - PRE_K, 1), f32)],
        axis=0)
    sc_fin = jnp.concatenate(
        [jnp.where(sc == -jnp.inf, 0.0, sc), jnp.zeros((KPAD - PRE_K, 1), f32)],
        axis=0)

    iota_col = jax.lax.broadcasted_iota(jnp.int32, (OUTPAD, 1), 0).astype(f32)
    outm = jnp.zeros((OUTPAD, 5), f32)
    outinf = jnp.zeros((OUTPAD, 1), f32)
    for r in range(NBLK):
        pos_r = pos[r:r + 1, :]
        p_r = (jnp.abs(iota_col - pos_r) < 0.25).astype(f32)
        data_r = jnp.concatenate(
            [tb_ref[r * BLK:(r + 1) * BLK, :],
             sc_fin[r * BLK:(r + 1) * BLK, :]], axis=1)
        outm = outm + jnp.dot(p_r, data_r, preferred_element_type=f32)
        outinf = outinf + jnp.dot(p_r, neg_inf_flag[r * BLK:(r + 1) * BLK, :],
                                  preferred_element_type=f32)
    out_scores = jnp.where(outinf > 0.5, -jnp.inf, outm[:, 4:5])
    out_ref[...] = jnp.concatenate([outm[:, 0:4], out_scores], axis=1)


def _decode(anchors, deltas, scores_row):
    return pl.pallas_call(
        _decode_kernel,
        out_shape=(
            jax.ShapeDtypeStruct((N_ANCHORS, 4), jnp.float32),
            jax.ShapeDtypeStruct((1, N_ANCHORS), jnp.float32),
        ),
    )(anchors, deltas, scores_row)


def _nms_select(top_boxes, top_scores_col):
    return pl.pallas_call(
        _nms_kernel,
        out_shape=jax.ShapeDtypeStruct((OUTPAD, 5), jnp.float32),
        scratch_shapes=[
            pltpu.VMEM((KPAD, 4), jnp.float32),
            pltpu.VMEM((4, KPAD), jnp.float32),
            pltpu.VMEM((BLK, BLK), jnp.float32),
        ],
    )(top_boxes, top_scores_col)


def kernel(anchors, objectness_logits, anchor_deltas):
    boxes, masked = _decode(anchors, anchor_deltas[0], objectness_logits)
    top_scores, top_idx = jax.lax.top_k(masked[0], PRE_K)
    top_boxes = boxes[top_idx]
    out = _nms_select(top_boxes, top_scores[:, None])
    return out[:POST_K]


# speedup vs baseline: 1.2643x; 1.2643x over previous
import math

import jax
import jax.numpy as jnp
from jax.experimental import pallas as pl
from jax.experimental.pallas import tpu as pltpu

N_ANCHORS = 20000
NPAD = 20480
ROWS = 160
PRE_K = 2000
KPAD = 2048
BLK = 256
NBLK = KPAD // BLK
POST_K = 1000
OUTPAD = 1024
NMS_T = 0.7
IMG = 1024.0
CLAMP = math.log(1000.0 / 16.0)


def _decode_kernel(a_ref, d_ref, s_ref, boxes_ref, ms_ref):
    x1a, y1a, x2a, y2a = a_ref[0], a_ref[1], a_ref[2], a_ref[3]
    dx, dy, dw, dh = d_ref[0], d_ref[1], d_ref[2], d_ref[3]
    w = x2a - x1a
    h = y2a - y1a
    cx = x1a + 0.5 * w
    cy = y1a + 0.5 * h
    dw = jnp.minimum(dw, CLAMP)
    dh = jnp.minimum(dh, CLAMP)
    px = dx * w + cx
    py = dy * h + cy
    pw = jnp.exp(dw) * w
    ph = jnp.exp(dh) * h
    x1 = jnp.clip(px - 0.5 * pw, 0.0, IMG)
    y1 = jnp.clip(py - 0.5 * ph, 0.0, IMG)
    x2 = jnp.clip(px + 0.5 * pw, 0.0, IMG)
    y2 = jnp.clip(py + 0.5 * ph, 0.0, IMG)
    boxes_ref[0] = x1
    boxes_ref[1] = y1
    boxes_ref[2] = x2
    boxes_ref[3] = y2
    valid = ((x2 - x1) > 0.0) & ((y2 - y1) > 0.0)
    gi = (jax.lax.broadcasted_iota(jnp.int32, (ROWS, 128), 0) * 128
          + jax.lax.broadcasted_iota(jnp.int32, (ROWS, 128), 1))
    ok = valid & (gi < N_ANCHORS)
    ms_ref[...] = jnp.where(ok, s_ref[...], -jnp.inf)


def _iou_block(cols, rows):
    x1i, y1i, x2i, y2i = cols
    x1j, y1j, x2j, y2j = rows
    ai = (x2i - x1i) * (y2i - y1i)
    aj = (x2j - x1j) * (y2j - y1j)
    wx = jnp.clip(jnp.minimum(x2i, x2j) - jnp.maximum(x1i, x1j), 0.0, None)
    wy = jnp.clip(jnp.minimum(y2i, y2j) - jnp.maximum(y1i, y1j), 0.0, None)
    inter = wx * wy
    union = ai + aj - inter
    return inter / jnp.maximum(union, 1e-9)


def _nms_kernel(in_tb_ref, in_sc_ref, out_ref, tb_ref, tbt_ref, over_ref):
    f32 = jnp.float32
    tb_ref[0:PRE_K, :] = in_tb_ref[...]
    tb_ref[PRE_K:KPAD, :] = jnp.zeros((KPAD - PRE_K, 4), f32)
    tbt_ref[...] = jnp.transpose(tb_ref[...])

    upper_s = (jax.lax.broadcasted_iota(jnp.int32, (BLK, BLK), 0)
               < jax.lax.broadcasted_iota(jnp.int32, (BLK, BLK), 1)).astype(f32)

    def cols_of(i):
        return tuple(tb_ref[i * BLK:(i + 1) * BLK, c:c + 1] for c in range(4))

    def rows_of(j):
        return tuple(tbt_ref[c:c + 1, j * BLK:(j + 1) * BLK] for c in range(4))

    keep_blocks = []
    for j in range(NBLK):
        rows_j = rows_of(j)
        supp = jnp.zeros((1, BLK), f32)
        for i in range(j):
            over_ij = (_iou_block(cols_of(i), rows_j) > NMS_T).astype(f32)
            supp = supp + jnp.dot(keep_blocks[i], over_ij,
                                  preferred_element_type=f32)
        over_ref[...] = ((_iou_block(cols_of(j), rows_j) > NMS_T).astype(f32)
                         * upper_s)

        def fix_body(carry):
            kb, _ = carry
            s2 = supp + jnp.dot(kb, over_ref[...], preferred_element_type=f32)
            kb2 = (s2 < 0.5).astype(f32)
            return kb2, jnp.any(kb2 != kb)

        keep_j, _ = jax.lax.while_loop(
            lambda c: c[1], fix_body,
            ((supp < 0.5).astype(f32), jnp.bool_(True)))
        keep_blocks.append(keep_j)

    keep = jnp.concatenate(keep_blocks, axis=0)
    gi = (jax.lax.broadcasted_iota(jnp.int32, (NBLK, BLK), 0) * BLK
          + jax.lax.broadcasted_iota(jnp.int32, (NBLK, BLK), 1))
    real = (gi < PRE_K).astype(f32)
    keep = keep * real
    nonkeep = real - keep

    upper = (jax.lax.broadcasted_iota(jnp.int32, (BLK, BLK), 0)
             <= jax.lax.broadcasted_iota(jnp.int32, (BLK, BLK), 1)).astype(f32)
    incl_k = jnp.dot(keep, upper, preferred_element_type=f32)
    incl_n = jnp.dot(nonkeep, upper, preferred_element_type=f32)

    def row_offsets(incl):
        offs = [jnp.zeros((1, 1), f32)]
        for r in range(1, NBLK):
            offs.append(offs[-1] + incl[r - 1:r, BLK - 1:BLK])
        return jnp.concatenate(offs, axis=0)

    offs_k = row_offsets(incl_k)
    offs_n = row_offsets(incl_n)
    n_kept = offs_k[NBLK - 1:NBLK, 0:1] + incl_k[NBLK - 1:NBLK, BLK - 1:BLK]
    excl_k = incl_k + offs_k - keep
    excl_n = incl_n + offs_n - nonkeep
    pos = jnp.where(keep > 0.5, excl_k, n_kept + excl_n)
    pos = jnp.where(gi < PRE_K, pos, 2.0 * OUTPAD)

    sc = in_sc_ref[...]
    neg_inf_flag = jnp.concatenate(
        [(sc == -jnp.inf).astype(f32), jnp.ones((KPAD - PRE_K, 1), f32)],
        axis=0)
    sc_fin = jnp.concatenate(
        [jnp.where(sc == -jnp.inf, 0.0, sc), jnp.zeros((KPAD - PRE_K, 1), f32)],
        axis=0)

    iota_col = jax.lax.broadcasted_iota(jnp.int32, (OUTPAD, 1), 0).astype(f32)
    outm = jnp.zeros((OUTPAD, 5), f32)
    outinf = jnp.zeros((OUTPAD, 1), f32)
    for r in range(NBLK):
        pos_r = pos[r:r + 1, :]
        p_r = (jnp.abs(iota_col - pos_r) < 0.25).astype(f32)
        data_r = jnp.concatenate(
            [tb_ref[r * BLK:(r + 1) * BLK, :],
             sc_fin[r * BLK:(r + 1) * BLK, :]], axis=1)
        outm = outm + jnp.dot(p_r, data_r, preferred_element_type=f32)
        outinf = outinf + jnp.dot(p_r, neg_inf_flag[r * BLK:(r + 1) * BLK, :],
                                  preferred_element_type=f32)
    out_scores = jnp.where(outinf > 0.5, -jnp.inf, outm[:, 4:5])
    out_ref[...] = jnp.concatenate([outm[:, 0:4], out_scores], axis=1)


def _decode(anchors_r, deltas_r, scores_r):
    return pl.pallas_call(
        _decode_kernel,
        out_shape=(
            jax.ShapeDtypeStruct((4, ROWS, 128), jnp.float32),
            jax.ShapeDtypeStruct((ROWS, 128), jnp.float32),
        ),
    )(anchors_r, deltas_r, scores_r)


def _nms_select(top_boxes, top_scores_col):
    return pl.pallas_call(
        _nms_kernel,
        out_shape=jax.ShapeDtypeStruct((OUTPAD, 5), jnp.float32),
        scratch_shapes=[
            pltpu.VMEM((KPAD, 4), jnp.float32),
            pltpu.VMEM((4, KPAD), jnp.float32),
            pltpu.VMEM((BLK, BLK), jnp.float32),
        ],
    )(top_boxes, top_scores_col)


def kernel(anchors, objectness_logits, anchor_deltas):
    scores = objectness_logits[0]
    deltas = anchor_deltas[0]
    pad = NPAD - N_ANCHORS
    anchors_r = jnp.pad(anchors, ((0, pad), (0, 0))).T.reshape(4, ROWS, 128)
    deltas_r = jnp.pad(deltas, ((0, pad), (0, 0))).T.reshape(4, ROWS, 128)
    scores_r = jnp.pad(scores, (0, pad)).reshape(ROWS, 128)

    boxes_r, masked_r = _decode(anchors_r, deltas_r, scores_r)
    boxes = boxes_r.reshape(4, NPAD).T
    masked = masked_r.reshape(NPAD)

    top_scores, top_idx = jax.lax.top_k(masked, PRE_K)
    top_boxes = boxes[top_idx]
    out = _nms_select(top_boxes, top_scores[:, None])
    return out[:POST_K]


# speedup vs baseline: 1.2785x; 1.0113x over previous
import math

import jax
import jax.numpy as jnp
from jax.experimental import pallas as pl
from jax.experimental.pallas import tpu as pltpu

N_ANCHORS = 20000
NPAD = 20480
ROWS = 160
PRE_K = 2000
KPAD = 2048
BLK = 256
NBLK = KPAD // BLK
POST_K = 1000
OUTPAD = 1024
NMS_T = 0.7
IMG = 1024.0
CLAMP = math.log(1000.0 / 16.0)


def _decode_kernel(ad_ref, s_ref, boxes_ref, ms_ref):
    x1a, y1a, x2a, y2a = ad_ref[0], ad_ref[1], ad_ref[2], ad_ref[3]
    dx, dy, dw, dh = ad_ref[4], ad_ref[5], ad_ref[6], ad_ref[7]
    w = x2a - x1a
    h = y2a - y1a
    cx = x1a + 0.5 * w
    cy = y1a + 0.5 * h
    dw = jnp.minimum(dw, CLAMP)
    dh = jnp.minimum(dh, CLAMP)
    px = dx * w + cx
    py = dy * h + cy
    pw = jnp.exp(dw) * w
    ph = jnp.exp(dh) * h
    x1 = jnp.clip(px - 0.5 * pw, 0.0, IMG)
    y1 = jnp.clip(py - 0.5 * ph, 0.0, IMG)
    x2 = jnp.clip(px + 0.5 * pw, 0.0, IMG)
    y2 = jnp.clip(py + 0.5 * ph, 0.0, IMG)
    boxes_ref[0] = x1
    boxes_ref[1] = y1
    boxes_ref[2] = x2
    boxes_ref[3] = y2
    valid = ((x2 - x1) > 0.0) & ((y2 - y1) > 0.0)
    gi = (jax.lax.broadcasted_iota(jnp.int32, (ROWS, 128), 0) * 128
          + jax.lax.broadcasted_iota(jnp.int32, (ROWS, 128), 1))
    ok = valid & (gi < N_ANCHORS)
    ms_ref[...] = jnp.where(ok, s_ref[...], -jnp.inf)


def _iou_block(cols, rows):
    x1i, y1i, x2i, y2i = cols
    x1j, y1j, x2j, y2j = rows
    ai = (x2i - x1i) * (y2i - y1i)
    aj = (x2j - x1j) * (y2j - y1j)
    wx = jnp.clip(jnp.minimum(x2i, x2j) - jnp.maximum(x1i, x1j), 0.0, None)
    wy = jnp.clip(jnp.minimum(y2i, y2j) - jnp.maximum(y1i, y1j), 0.0, None)
    inter = wx * wy
    union = ai + aj - inter
    return inter / jnp.maximum(union, 1e-9)


def _nms_kernel(in_tb_ref, in_sc_ref, out_ref, tb_ref, tbt_ref, over_ref):
    f32 = jnp.float32
    tb_ref[0:PRE_K, :] = in_tb_ref[...]
    tb_ref[PRE_K:KPAD, :] = jnp.zeros((KPAD - PRE_K, 4), f32)
    tbt_ref[...] = jnp.transpose(tb_ref[...])

    upper_s = (jax.lax.broadcasted_iota(jnp.int32, (BLK, BLK), 0)
               < jax.lax.broadcasted_iota(jnp.int32, (BLK, BLK), 1)).astype(f32)

    def cols_of(i):
        return tuple(tb_ref[i * BLK:(i + 1) * BLK, c:c + 1] for c in range(4))

    def rows_of(j):
        return tuple(tbt_ref[c:c + 1, j * BLK:(j + 1) * BLK] for c in range(4))

    keep_blocks = []
    for j in range(NBLK):
        rows_j = rows_of(j)
        supp = jnp.zeros((1, BLK), f32)
        for i in range(j):
            over_ij = (_iou_block(cols_of(i), rows_j) > NMS_T).astype(f32)
            supp = supp + jnp.dot(keep_blocks[i], over_ij,
                                  preferred_element_type=f32)
        over_ref[...] = ((_iou_block(cols_of(j), rows_j) > NMS_T).astype(f32)
                         * upper_s)

        def fix_body(carry):
            kb, _ = carry
            s2 = supp + jnp.dot(kb, over_ref[...], preferred_element_type=f32)
            kb2 = (s2 < 0.5).astype(f32)
            return kb2, jnp.any(kb2 != kb)

        keep_j, _ = jax.lax.while_loop(
            lambda c: c[1], fix_body,
            ((supp < 0.5).astype(f32), jnp.bool_(True)))
        keep_blocks.append(keep_j)

    keep = jnp.concatenate(keep_blocks, axis=0)
    gi = (jax.lax.broadcasted_iota(jnp.int32, (NBLK, BLK), 0) * BLK
          + jax.lax.broadcasted_iota(jnp.int32, (NBLK, BLK), 1))
    real = (gi < PRE_K).astype(f32)
    keep = keep * real
    nonkeep = real - keep

    upper = (jax.lax.broadcasted_iota(jnp.int32, (BLK, BLK), 0)
             <= jax.lax.broadcasted_iota(jnp.int32, (BLK, BLK), 1)).astype(f32)
    incl_k = jnp.dot(keep, upper, preferred_element_type=f32)
    incl_n = jnp.dot(nonkeep, upper, preferred_element_type=f32)

    def row_offsets(incl):
        offs = [jnp.zeros((1, 1), f32)]
        for r in range(1, NBLK):
            offs.append(offs[-1] + incl[r - 1:r, BLK - 1:BLK])
        return jnp.concatenate(offs, axis=0)

    offs_k = row_offsets(incl_k)
    offs_n = row_offsets(incl_n)
    n_kept = offs_k[NBLK - 1:NBLK, 0:1] + incl_k[NBLK - 1:NBLK, BLK - 1:BLK]
    excl_k = incl_k + offs_k - keep
    excl_n = incl_n + offs_n - nonkeep
    pos = jnp.where(keep > 0.5, excl_k, n_kept + excl_n)
    pos = jnp.where(gi < PRE_K, pos, 2.0 * OUTPAD)

    sc = in_sc_ref[...]
    neg_inf_flag = jnp.concatenate(
        [(sc == -jnp.inf).astype(f32), jnp.ones((KPAD - PRE_K, 1), f32)],
        axis=0)
    sc_fin = jnp.concatenate(
        [jnp.where(sc == -jnp.inf, 0.0, sc), jnp.zeros((KPAD - PRE_K, 1), f32)],
        axis=0)

    iota_col = jax.lax.broadcasted_iota(jnp.int32, (OUTPAD, 1), 0).astype(f32)
    outm = jnp.zeros((OUTPAD, 5), f32)
    outinf = jnp.zeros((OUTPAD, 1), f32)
    for r in range(NBLK):
        pos_r = pos[r:r + 1, :]
        p_r = (jnp.abs(iota_col - pos_r) < 0.25).astype(f32)
        data_r = jnp.concatenate(
            [tb_ref[r * BLK:(r + 1) * BLK, :],
             sc_fin[r * BLK:(r + 1) * BLK, :]], axis=1)
        outm = outm + jnp.dot(p_r, data_r, preferred_element_type=f32)
        outinf = outinf + jnp.dot(p_r, neg_inf_flag[r * BLK:(r + 1) * BLK, :],
                                  preferred_element_type=f32)
    out_scores = jnp.where(outinf > 0.5, -jnp.inf, outm[:, 4:5])
    out_ref[...] = jnp.concatenate([outm[:, 0:4], out_scores], axis=1)


def _decode(ad_r, scores_r):
    return pl.pallas_call(
        _decode_kernel,
        out_shape=(
            jax.ShapeDtypeStruct((4, ROWS, 128), jnp.float32),
            jax.ShapeDtypeStruct((ROWS, 128), jnp.float32),
        ),
    )(ad_r, scores_r)


def _nms_select(top_boxes, top_scores_col):
    return pl.pallas_call(
        _nms_kernel,
        out_shape=jax.ShapeDtypeStruct((OUTPAD, 5), jnp.float32),
        scratch_shapes=[
            pltpu.VMEM((KPAD, 4), jnp.float32),
            pltpu.VMEM((4, KPAD), jnp.float32),
            pltpu.VMEM((BLK, BLK), jnp.float32),
        ],
    )(top_boxes, top_scores_col)


def kernel(anchors, objectness_logits, anchor_deltas):
    scores = objectness_logits[0]
    deltas = anchor_deltas[0]
    pad = NPAD - N_ANCHORS
    ad = jnp.concatenate([anchors, deltas], axis=1)
    ad_r = jnp.pad(ad, ((0, pad), (0, 0))).T.reshape(8, ROWS, 128)
    scores_r = jnp.pad(scores, (0, pad)).reshape(ROWS, 128)

    boxes_r, masked_r = _decode(ad_r, scores_r)
    boxes = boxes_r.reshape(4, NPAD).T
    masked = masked_r.reshape(NPAD)

    top_scores, top_idx = jax.lax.top_k(masked, PRE_K)
    top_boxes = boxes[top_idx]
    out = _nms_select(top_boxes, top_scores[:, None])
    return out[:POST_K]
